# Initial kernel scaffold; baseline (speedup 1.0000x reference)
import jax
import jax.numpy as jnp
from jax.experimental import pallas as pl


def kernel(x, edge_index, weight, bias):
    raise NotImplementedError("write your pallas kernel here")


# speedup vs baseline: 131.1453x; 131.1453x over previous
import functools

import jax
import jax.numpy as jnp
from jax import lax
from jax.experimental import pallas as pl
from jax.experimental.pallas import tpu as pltpu
from jax.experimental.pallas import tpu_sc as plsc

N = 50000
E = 800000
D = 64
G = 16
NC = 2
NS = 16
L = 16

CE = 1280
NSUB = CE // 128
NCHUNK = E // CE

CS = 256
SSUB = CS // 128
SCHUNK = E // CS

NH = N // NC
DUMP = NH
ACC_ROWS = NH + 8
WB = 1568
WB_LAST = NH - 15 * WB

DEG_WB = 3136
DEG_WB_LAST = N - 15 * DEG_WB

_mesh = plsc.VectorSubcoreMesh(
    core_axis_name="c", subcore_axis_name="s", num_cores=NC, num_subcores=NS)


def _deg_body(edge_hbm, zrow_hbm, out_hbm, rbuf, cbuf, vbuf, zv, acc, sem):
  c = lax.axis_index("c")
  s = lax.axis_index("s")

  pltpu.sync_copy(zrow_hbm, zv)

  @pl.when(s < 15)
  def _():
    pltpu.sync_copy(zv, acc.at[pl.ds(s * DEG_WB, DEG_WB)])

  @pl.when(s == 15)
  def _():
    pltpu.sync_copy(zv.at[pl.ds(0, DEG_WB_LAST)],
                    acc.at[pl.ds(s * DEG_WB, DEG_WB_LAST)])

  plsc.subcore_barrier()

  w = c * NS + s
  nck = (NCHUNK - w + NC * NS - 1) // (NC * NS)

  def body(j, carry):
    ck = w + j * (NC * NS)
    off = ck * CE
    descs = []
    for t in range(NSUB):
      descs.append(pltpu.async_copy(
          edge_hbm.at[0, pl.ds(off + t * 128, 128)], rbuf.at[t], sem))
    descs.append(pltpu.async_copy(edge_hbm.at[1, pl.ds(off, CE)], cbuf, sem))
    for d in descs:
      d.wait()
    for i in range(CE // L):
      r16 = rbuf[i // 8, pl.ds((i % 8) * L, L)]
      c16 = cbuf[pl.ds(i * L, L)]
      v16 = jnp.where(r16 != c16, jnp.float32(1.0), jnp.float32(0.0))
      vbuf[i // 8, pl.ds((i % 8) * L, L)] = v16
    for t in range(NSUB):
      pltpu.sync_copy(vbuf.at[t], acc.at[rbuf.at[t]], add=True)
    return carry

  lax.fori_loop(0, nck, body, 0)
  plsc.subcore_barrier()

  @pl.when(s < 15)
  def _():
    pltpu.sync_copy(acc.at[pl.ds(s * DEG_WB, DEG_WB)], zv)
    pltpu.sync_copy(zv, out_hbm.at[pl.ds(c * N + s * DEG_WB, DEG_WB)])

  @pl.when(s == 15)
  def _():
    pltpu.sync_copy(acc.at[pl.ds(s * DEG_WB, DEG_WB_LAST)],
                    zv.at[pl.ds(0, DEG_WB_LAST)])
    pltpu.sync_copy(zv.at[pl.ds(0, DEG_WB_LAST)],
                    out_hbm.at[pl.ds(c * N + s * DEG_WB, DEG_WB_LAST)])


_deg_call = pl.kernel(
    _deg_body,
    out_type=jax.ShapeDtypeStruct((NC * N,), jnp.float32),
    mesh=_mesh,
    scratch_types=[
        pltpu.VMEM((NSUB, 128), jnp.int32),
        pltpu.VMEM((CE,), jnp.int32),
        pltpu.VMEM((NSUB, 128), jnp.float32),
        pltpu.VMEM((DEG_WB,), jnp.float32),
        pltpu.VMEM_SHARED((N,), jnp.float32),
        pltpu.SemaphoreType.DMA,
    ],
)


def _segments(total, step):
  segs = []
  off = 0
  while off < total:
    seg = min(step, total - off)
    segs.append((off, seg))
    off += seg
  return segs


def _spmm_body(edge_hbm, z_hbm, zblk_hbm, out_hbm,
               rbuf, cbuf, libuf, rows, acc, sem):
  c = lax.axis_index("c")
  s = lax.axis_index("s")
  base = c * NH

  pltpu.sync_copy(zblk_hbm, rows)

  @pl.when(s < 15)
  def _():
    for off, seg in _segments(WB, CS):
      pltpu.sync_copy(rows.at[pl.ds(0, seg)],
                      acc.at[pl.ds(s * WB + off, seg)])

  @pl.when(s == 15)
  def _():
    for off, seg in _segments(WB_LAST, CS):
      pltpu.sync_copy(rows.at[pl.ds(0, seg)],
                      acc.at[pl.ds(s * WB + off, seg)])

  @pl.when(s == 0)
  def _():
    pltpu.sync_copy(rows.at[pl.ds(0, ACC_ROWS - NH)],
                    acc.at[pl.ds(NH, ACC_ROWS - NH)])

  plsc.subcore_barrier()

  nck = (SCHUNK - s + NS - 1) // NS

  def body(j, carry):
    ck = s + j * NS
    off = ck * CS
    d0 = pltpu.async_copy(edge_hbm.at[0, pl.ds(off, CS)], rbuf, sem)
    d1 = pltpu.async_copy(edge_hbm.at[1, pl.ds(off, CS)], cbuf, sem)
    d0.wait()
    d1.wait()
    gds = []
    for t in range(SSUB):
      gds.append(pltpu.async_copy(
          z_hbm.at[cbuf.at[pl.ds(t * 128, 128)]],
          rows.at[pl.ds(t * 128, 128)], sem))
    for i in range(CS // L):
      r16 = rbuf[pl.ds(i * L, L)]
      c16 = cbuf[pl.ds(i * L, L)]
      keep = (r16 != c16) & (r16 >= base) & (r16 < base + NH)
      li = jnp.where(keep, r16 - base, jnp.int32(DUMP))
      libuf[i // 8, pl.ds((i % 8) * L, L)] = li
    for d in gds:
      d.wait()
    for t in range(SSUB):
      pltpu.sync_copy(rows.at[pl.ds(t * 128, 128)],
                      acc.at[libuf.at[t]], add=True)
    return carry

  lax.fori_loop(0, nck, body, 0)
  plsc.subcore_barrier()

  @pl.when(s < 15)
  def _():
    for off, seg in _segments(WB, CS):
      pltpu.sync_copy(acc.at[pl.ds(s * WB + off, seg)],
                      rows.at[pl.ds(0, seg)])
      pltpu.sync_copy(rows.at[pl.ds(0, seg)],
                      out_hbm.at[pl.ds(base + s * WB + off, seg)])

  @pl.when(s == 15)
  def _():
    for off, seg in _segments(WB_LAST, CS):
      pltpu.sync_copy(acc.at[pl.ds(s * WB + off, seg)],
                      rows.at[pl.ds(0, seg)])
      pltpu.sync_copy(rows.at[pl.ds(0, seg)],
                      out_hbm.at[pl.ds(base + s * WB + off, seg)])


_spmm_call = pl.kernel(
    _spmm_body,
    out_type=jax.ShapeDtypeStruct((N, D), jnp.float32),
    mesh=_mesh,
    compiler_params=pltpu.CompilerParams(use_tc_tiling_on_sc=False),
    scratch_types=[
        pltpu.VMEM((CS,), jnp.int32),
        pltpu.VMEM((CS,), jnp.int32),
        pltpu.VMEM((SSUB, 128), jnp.int32),
        pltpu.VMEM((CS, D), jnp.float32),
        pltpu.VMEM_SHARED((ACC_ROWS, D), jnp.float32),
        pltpu.SemaphoreType.DMA,
    ],
)


BR = 2000
GRID = N // BR


def _prep_body(x_ref, p0_ref, p1_ref, z_ref, dis_ref):
  deg = p0_ref[...] + p1_ref[...]
  dis = jnp.where(deg > 0, lax.rsqrt(deg), jnp.float32(0.0))
  dis_ref[...] = dis
  z_ref[...] = dis * x_ref[...]


_prep_call = pl.pallas_call(
    _prep_body,
    grid=(GRID,),
    in_specs=[
        pl.BlockSpec((BR, D), lambda i: (i, 0)),
        pl.BlockSpec((BR, 1), lambda i: (i, 0)),
        pl.BlockSpec((BR, 1), lambda i: (i, 0)),
    ],
    out_specs=[
        pl.BlockSpec((BR, D), lambda i: (i, 0)),
        pl.BlockSpec((BR, 1), lambda i: (i, 0)),
    ],
    out_shape=[
        jax.ShapeDtypeStruct((N, D), jnp.float32),
        jax.ShapeDtypeStruct((N, 1), jnp.float32),
    ],
)


def _scale2_body(s1_ref, dis_ref, z2_ref):
  dis = dis_ref[...]
  z2_ref[...] = -(dis * dis) * s1_ref[...]


_scale2_call = pl.pallas_call(
    _scale2_body,
    grid=(GRID,),
    in_specs=[
        pl.BlockSpec((BR, D), lambda i: (i, 0)),
        pl.BlockSpec((BR, 1), lambda i: (i, 0)),
    ],
    out_specs=pl.BlockSpec((BR, D), lambda i: (i, 0)),
    out_shape=jax.ShapeDtypeStruct((N, D), jnp.float32),
)


def _final_body(x_ref, s1_ref, s2_ref, dis_ref, w_ref, b_ref, o_ref):
  x = x_ref[...]
  dis = dis_ref[...]
  tx1 = -dis * s1_ref[...]
  tx2 = jnp.float32(-2.0) * dis * s2_ref[...] - x
  w = w_ref[...]
  o = jnp.dot(x, w[0], preferred_element_type=jnp.float32)
  o += jnp.dot(tx1, w[1], preferred_element_type=jnp.float32)
  o += jnp.dot(tx2, w[2], preferred_element_type=jnp.float32)
  o_ref[...] = o + b_ref[...]


_final_call = pl.pallas_call(
    _final_body,
    grid=(GRID,),
    in_specs=[
        pl.BlockSpec((BR, D), lambda i: (i, 0)),
        pl.BlockSpec((BR, D), lambda i: (i, 0)),
        pl.BlockSpec((BR, D), lambda i: (i, 0)),
        pl.BlockSpec((BR, 1), lambda i: (i, 0)),
        pl.BlockSpec((3, D, G), lambda i: (0, 0, 0)),
        pl.BlockSpec((1, G), lambda i: (0, 0)),
    ],
    out_specs=pl.BlockSpec((BR, G), lambda i: (i, 0)),
    out_shape=jax.ShapeDtypeStruct((N, G), jnp.float32),
)


@jax.jit
def kernel(x, edge_index, weight, bias):
  n, h, f, q = x.shape
  g = weight.shape[-1]
  x2d = x.reshape(n, h * f * q)
  w3 = weight.reshape(weight.shape[0], h * f, g)
  b2 = bias.reshape(1, g)

  zrow = jnp.zeros((DEG_WB,), jnp.float32)
  zblk = jnp.zeros((CS, D), jnp.float32)

  parts = _deg_call(edge_index, zrow)
  p0 = parts[:N].reshape(n, 1)
  p1 = parts[N:].reshape(n, 1)
  z1, dis = _prep_call(x2d, p0, p1)
  s1 = _spmm_call(edge_index, z1, zblk)
  z2 = _scale2_call(s1, dis)
  s2 = _spmm_call(edge_index, z2, zblk)
  out = _final_call(x2d, s1, s2, dis, w3, b2)
  return out.reshape(n, 1, g, q)


# speedup vs baseline: 137.1005x; 1.0454x over previous
import functools

import jax
import jax.numpy as jnp
from jax import lax
from jax.experimental import pallas as pl
from jax.experimental.pallas import tpu as pltpu
from jax.experimental.pallas import tpu_sc as plsc

N = 50000
E = 800000
D = 64
G = 16
NC = 2
NS = 16
L = 16

CE = 1280
NSUB = CE // 128
NCHUNK = E // CE

EB = 1280
BSUB = EB // 128
NBLK = E // EB
NSLOT = 3

NH = N // NC
DUMP = NH
ACC_ROWS = NH + 8
WB = 1568
WB_LAST = NH - 15 * WB

DEG_WB = 3136
DEG_WB_LAST = N - 15 * DEG_WB

_mesh = plsc.VectorSubcoreMesh(
    core_axis_name="c", subcore_axis_name="s", num_cores=NC, num_subcores=NS)


def _deg_body(edge_hbm, zrow_hbm, out_hbm, rbuf, cbuf, vbuf, zv, acc, sem):
  c = lax.axis_index("c")
  s = lax.axis_index("s")

  pltpu.sync_copy(zrow_hbm, zv)

  @pl.when(s < 15)
  def _():
    pltpu.sync_copy(zv, acc.at[pl.ds(s * DEG_WB, DEG_WB)])

  @pl.when(s == 15)
  def _():
    pltpu.sync_copy(zv.at[pl.ds(0, DEG_WB_LAST)],
                    acc.at[pl.ds(s * DEG_WB, DEG_WB_LAST)])

  plsc.subcore_barrier()

  w = c * NS + s
  nck = (NCHUNK - w + NC * NS - 1) // (NC * NS)

  def body(j, carry):
    ck = w + j * (NC * NS)
    off = ck * CE
    descs = []
    for t in range(NSUB):
      descs.append(pltpu.async_copy(
          edge_hbm.at[0, pl.ds(off + t * 128, 128)], rbuf.at[t], sem))
    descs.append(pltpu.async_copy(edge_hbm.at[1, pl.ds(off, CE)], cbuf, sem))
    for d in descs:
      d.wait()
    for i in range(CE // L):
      r16 = rbuf[i // 8, pl.ds((i % 8) * L, L)]
      c16 = cbuf[pl.ds(i * L, L)]
      v16 = jnp.where(r16 != c16, jnp.float32(1.0), jnp.float32(0.0))
      vbuf[i // 8, pl.ds((i % 8) * L, L)] = v16
    for t in range(NSUB):
      pltpu.sync_copy(vbuf.at[t], acc.at[rbuf.at[t]], add=True)
    return carry

  lax.fori_loop(0, nck, body, 0)
  plsc.subcore_barrier()

  @pl.when(s < 15)
  def _():
    pltpu.sync_copy(acc.at[pl.ds(s * DEG_WB, DEG_WB)], zv)
    pltpu.sync_copy(zv, out_hbm.at[pl.ds(c * N + s * DEG_WB, DEG_WB)])

  @pl.when(s == 15)
  def _():
    pltpu.sync_copy(acc.at[pl.ds(s * DEG_WB, DEG_WB_LAST)],
                    zv.at[pl.ds(0, DEG_WB_LAST)])
    pltpu.sync_copy(zv.at[pl.ds(0, DEG_WB_LAST)],
                    out_hbm.at[pl.ds(c * N + s * DEG_WB, DEG_WB_LAST)])


_deg_call = pl.kernel(
    _deg_body,
    out_type=jax.ShapeDtypeStruct((NC * N,), jnp.float32),
    mesh=_mesh,
    scratch_types=[
        pltpu.VMEM((NSUB, 128), jnp.int32),
        pltpu.VMEM((CE,), jnp.int32),
        pltpu.VMEM((NSUB, 128), jnp.float32),
        pltpu.VMEM((DEG_WB,), jnp.float32),
        pltpu.VMEM_SHARED((N,), jnp.float32),
        pltpu.SemaphoreType.DMA,
    ],
)


def _segments(total, step):
  segs = []
  off = 0
  while off < total:
    seg = min(step, total - off)
    segs.append((off, seg))
    off += seg
  return segs


def _spmm_body(edge_hbm, z_hbm, zblk_hbm, out_hbm,
               rbuf, cbuf, libuf, slots, acc, esem, gsem, ssem, wsem):
  c = lax.axis_index("c")
  s = lax.axis_index("s")
  base = c * NH

  pltpu.sync_copy(zblk_hbm, slots.at[0])

  def zero_fill(total):
    zds = []
    for off, seg in _segments(total, 128):
      zds.append(pltpu.async_copy(slots.at[0].at[pl.ds(0, seg)],
                                  acc.at[pl.ds(s * WB + off, seg)], wsem))
    for d in zds:
      d.wait()

  @pl.when(s < 15)
  def _():
    zero_fill(WB)

  @pl.when(s == 15)
  def _():
    zero_fill(WB_LAST)

  @pl.when(s == 0)
  def _():
    pltpu.sync_copy(slots.at[0].at[pl.ds(0, ACC_ROWS - NH)],
                    acc.at[pl.ds(NH, ACC_ROWS - NH)])
  plsc.subcore_barrier()

  nblk = (NBLK - s + NS - 1) // NS

  def body(j, carry):
    blk = s + j * NS
    off = blk * EB
    d0 = pltpu.async_copy(edge_hbm.at[0, pl.ds(off, EB)], rbuf, esem)
    d1 = pltpu.async_copy(edge_hbm.at[1, pl.ds(off, EB)], cbuf, esem)
    d0.wait()
    d1.wait()
    gds, sds = [], []
    for t in range(NSLOT):
      gds.append(pltpu.async_copy(
          z_hbm.at[cbuf.at[pl.ds(t * 128, 128)]], slots.at[t], gsem))
    for i in range(EB // L):
      r16 = rbuf[pl.ds(i * L, L)]
      c16 = cbuf[pl.ds(i * L, L)]
      keep = (r16 != c16) & (r16 >= base) & (r16 < base + NH)
      li = jnp.where(keep, r16 - base, jnp.int32(DUMP))
      libuf[i // 8, pl.ds((i % 8) * L, L)] = li
    for t in range(BSUB):
      if t >= 1 and t + 2 < BSUB:
        sds[t - 1].wait()
        gds.append(pltpu.async_copy(
            z_hbm.at[cbuf.at[pl.ds((t + 2) * 128, 128)]],
            slots.at[(t + 2) % NSLOT], gsem))
      gds[t].wait()
      sd = pltpu.make_async_copy(slots.at[t % NSLOT],
                                 acc.at[libuf.at[t]], ssem)
      sd.start(add=True)
      sds.append(sd)
    for t in range(BSUB - 3, BSUB):
      sds[t].wait()
    return carry

  lax.fori_loop(0, nblk, body, 0)
  plsc.subcore_barrier()

  def writeback(total):
    wds = {}
    segs = _segments(total, 128)
    for k, (off, seg) in enumerate(segs):
      if k >= 2:
        wds[k - 2].wait()
      pltpu.sync_copy(acc.at[pl.ds(s * WB + off, seg)],
                      slots.at[k % 2].at[pl.ds(0, seg)])
      wds[k] = pltpu.async_copy(
          slots.at[k % 2].at[pl.ds(0, seg)],
          out_hbm.at[pl.ds(base + s * WB + off, seg)], wsem)
    for k in range(max(len(segs) - 2, 0), len(segs)):
      wds[k].wait()

  @pl.when(s < 15)
  def _():
    writeback(WB)

  @pl.when(s == 15)
  def _():
    writeback(WB_LAST)


_spmm_call = pl.kernel(
    _spmm_body,
    out_type=jax.ShapeDtypeStruct((N, D), jnp.float32),
    mesh=_mesh,
    compiler_params=pltpu.CompilerParams(use_tc_tiling_on_sc=False),
    scratch_types=[
        pltpu.VMEM((EB,), jnp.int32),
        pltpu.VMEM((EB,), jnp.int32),
        pltpu.VMEM((BSUB, 128), jnp.int32),
        pltpu.VMEM((NSLOT, 128, D), jnp.float32),
        pltpu.VMEM_SHARED((ACC_ROWS, D), jnp.float32),
        pltpu.SemaphoreType.DMA,
        pltpu.SemaphoreType.DMA,
        pltpu.SemaphoreType.DMA,
        pltpu.SemaphoreType.DMA,
    ],
)


BR = 2000
GRID = N // BR


def _prep_body(x_ref, p0_ref, p1_ref, z_ref, dis_ref):
  deg = p0_ref[...] + p1_ref[...]
  dis = jnp.where(deg > 0, lax.rsqrt(deg), jnp.float32(0.0))
  dis_ref[...] = dis
  z_ref[...] = dis * x_ref[...]


_prep_call = pl.pallas_call(
    _prep_body,
    grid=(GRID,),
    in_specs=[
        pl.BlockSpec((BR, D), lambda i: (i, 0)),
        pl.BlockSpec((BR, 1), lambda i: (i, 0)),
        pl.BlockSpec((BR, 1), lambda i: (i, 0)),
    ],
    out_specs=[
        pl.BlockSpec((BR, D), lambda i: (i, 0)),
        pl.BlockSpec((BR, 1), lambda i: (i, 0)),
    ],
    out_shape=[
        jax.ShapeDtypeStruct((N, D), jnp.float32),
        jax.ShapeDtypeStruct((N, 1), jnp.float32),
    ],
)


def _scale2_body(s1_ref, dis_ref, z2_ref):
  dis = dis_ref[...]
  z2_ref[...] = -(dis * dis) * s1_ref[...]


_scale2_call = pl.pallas_call(
    _scale2_body,
    grid=(GRID,),
    in_specs=[
        pl.BlockSpec((BR, D), lambda i: (i, 0)),
        pl.BlockSpec((BR, 1), lambda i: (i, 0)),
    ],
    out_specs=pl.BlockSpec((BR, D), lambda i: (i, 0)),
    out_shape=jax.ShapeDtypeStruct((N, D), jnp.float32),
)


def _final_body(x_ref, s1_ref, s2_ref, dis_ref, w_ref, b_ref, o_ref):
  x = x_ref[...]
  dis = dis_ref[...]
  tx1 = -dis * s1_ref[...]
  tx2 = jnp.float32(-2.0) * dis * s2_ref[...] - x
  w = w_ref[...]
  o = jnp.dot(x, w[0], preferred_element_type=jnp.float32)
  o += jnp.dot(tx1, w[1], preferred_element_type=jnp.float32)
  o += jnp.dot(tx2, w[2], preferred_element_type=jnp.float32)
  o_ref[...] = o + b_ref[...]


_final_call = pl.pallas_call(
    _final_body,
    grid=(GRID,),
    in_specs=[
        pl.BlockSpec((BR, D), lambda i: (i, 0)),
        pl.BlockSpec((BR, D), lambda i: (i, 0)),
        pl.BlockSpec((BR, D), lambda i: (i, 0)),
        pl.BlockSpec((BR, 1), lambda i: (i, 0)),
        pl.BlockSpec((3, D, G), lambda i: (0, 0, 0)),
        pl.BlockSpec((1, G), lambda i: (0, 0)),
    ],
    out_specs=pl.BlockSpec((BR, G), lambda i: (i, 0)),
    out_shape=jax.ShapeDtypeStruct((N, G), jnp.float32),
)


@jax.jit
def kernel(x, edge_index, weight, bias):
  n, h, f, q = x.shape
  g = weight.shape[-1]
  x2d = x.reshape(n, h * f * q)
  w3 = weight.reshape(weight.shape[0], h * f, g)
  b2 = bias.reshape(1, g)

  zrow = jnp.zeros((DEG_WB,), jnp.float32)
  zblk = jnp.zeros((128, D), jnp.float32)

  parts = _deg_call(edge_index, zrow)
  p0 = parts[:N].reshape(n, 1)
  p1 = parts[N:].reshape(n, 1)
  z1, dis = _prep_call(x2d, p0, p1)
  s1 = _spmm_call(edge_index, z1, zblk)
  z2 = _scale2_call(s1, dis)
  s2 = _spmm_call(edge_index, z2, zblk)
  out = _final_call(x2d, s1, s2, dis, w3, b2)
  return out.reshape(n, 1, g, q)


# speedup vs baseline: 239.3430x; 1.7457x over previous
import jax
import jax.numpy as jnp
from jax import lax
from jax.experimental import pallas as pl
from jax.experimental.pallas import tpu as pltpu
from jax.experimental.pallas import tpu_sc as plsc

N = 50000
E = 800000
D = 64
DH = D // 2
G = 16
NC = 2
NS = 16
L = 16

EB = 1280
BSUB = EB // 128
NBLK = E // EB
NSLOT = 4

DUMP = N
ACC_ROWS = N + 8
WBF = 3136
WBF_LAST = N - 15 * WBF

CE = 1280
NSUB = CE // 128
NCHUNK = E // CE

_mesh = plsc.VectorSubcoreMesh(
    core_axis_name="c", subcore_axis_name="s", num_cores=NC, num_subcores=NS)


def _segments(total, step):
  segs = []
  off = 0
  while off < total:
    seg = min(step, total - off)
    segs.append((off, seg))
    off += seg
  return segs


def _deg_body(edge_hbm, zrow_hbm, out_hbm, rbuf, cbuf, vbuf, zv, acc, sem):
  c = lax.axis_index("c")
  s = lax.axis_index("s")

  pltpu.sync_copy(zrow_hbm, zv)

  @pl.when(s < 15)
  def _():
    pltpu.sync_copy(zv, acc.at[pl.ds(s * WBF, WBF)])

  @pl.when(s == 15)
  def _():
    pltpu.sync_copy(zv.at[pl.ds(0, WBF_LAST)],
                    acc.at[pl.ds(s * WBF, WBF_LAST)])

  plsc.subcore_barrier()

  w = c * NS + s
  nck = (NCHUNK - w + NC * NS - 1) // (NC * NS)

  def body(j, carry):
    ck = w + j * (NC * NS)
    off = ck * CE
    descs = []
    for t in range(NSUB):
      descs.append(pltpu.async_copy(
          edge_hbm.at[0, pl.ds(off + t * 128, 128)], rbuf.at[t], sem))
    descs.append(pltpu.async_copy(edge_hbm.at[1, pl.ds(off, CE)], cbuf, sem))
    for d in descs:
      d.wait()
    for i in range(CE // L):
      r16 = rbuf[i // 8, pl.ds((i % 8) * L, L)]
      c16 = cbuf[pl.ds(i * L, L)]
      v16 = jnp.where(r16 != c16, jnp.float32(1.0), jnp.float32(0.0))
      vbuf[i // 8, pl.ds((i % 8) * L, L)] = v16
    for t in range(NSUB):
      pltpu.sync_copy(vbuf.at[t], acc.at[rbuf.at[t]], add=True)
    return carry

  lax.fori_loop(0, nck, body, 0)
  plsc.subcore_barrier()

  @pl.when(s < 15)
  def _():
    pltpu.sync_copy(acc.at[pl.ds(s * WBF, WBF)], zv)
    pltpu.sync_copy(zv, out_hbm.at[pl.ds(c * N + s * WBF, WBF)])

  @pl.when(s == 15)
  def _():
    pltpu.sync_copy(acc.at[pl.ds(s * WBF, WBF_LAST)],
                    zv.at[pl.ds(0, WBF_LAST)])
    pltpu.sync_copy(zv.at[pl.ds(0, WBF_LAST)],
                    out_hbm.at[pl.ds(c * N + s * WBF, WBF_LAST)])


_deg_call = pl.kernel(
    _deg_body,
    out_type=jax.ShapeDtypeStruct((NC * N,), jnp.float32),
    mesh=_mesh,
    scratch_types=[
        pltpu.VMEM((NSUB, 128), jnp.int32),
        pltpu.VMEM((CE,), jnp.int32),
        pltpu.VMEM((NSUB, 128), jnp.float32),
        pltpu.VMEM((WBF,), jnp.float32),
        pltpu.VMEM_SHARED((N,), jnp.float32),
        pltpu.SemaphoreType.DMA,
    ],
)


def _spmm_body(edge_hbm, zlo_hbm, zhi_hbm, zblk_hbm, outlo_hbm, outhi_hbm,
               rbuf, cbuf, libuf, slots, acc, esem, gsem, ssem, wsem):
  c = lax.axis_index("c")
  s = lax.axis_index("s")

  pltpu.sync_copy(zblk_hbm, slots.at[0])

  def zero_fill(total):
    zds = []
    for off, seg in _segments(total, 128):
      zds.append(pltpu.async_copy(slots.at[0].at[pl.ds(0, seg)],
                                  acc.at[pl.ds(s * WBF + off, seg)], wsem))
    for d in zds:
      d.wait()

  @pl.when(s < 15)
  def _():
    zero_fill(WBF)

  @pl.when(s == 15)
  def _():
    zero_fill(WBF_LAST)

  @pl.when(s == 0)
  def _():
    pltpu.sync_copy(slots.at[0].at[pl.ds(0, ACC_ROWS - N)],
                    acc.at[pl.ds(N, ACC_ROWS - N)])
  plsc.subcore_barrier()

  nblk = (NBLK - s + NS - 1) // NS

  def edge_loop(z_hbm):
    def body(j, carry):
      blk = s + j * NS
      off = blk * EB
      d0 = pltpu.async_copy(edge_hbm.at[0, pl.ds(off, EB)], rbuf, esem)
      d1 = pltpu.async_copy(edge_hbm.at[1, pl.ds(off, EB)], cbuf, esem)
      d0.wait()
      d1.wait()
      gds, sds = [], []
      for t in range(NSLOT):
        gds.append(pltpu.async_copy(
            z_hbm.at[cbuf.at[pl.ds(t * 128, 128)]], slots.at[t], gsem))
      for i in range(EB // L):
        r16 = rbuf[pl.ds(i * L, L)]
        c16 = cbuf[pl.ds(i * L, L)]
        li = jnp.where(r16 != c16, r16, jnp.int32(DUMP))
        libuf[i // 8, pl.ds((i % 8) * L, L)] = li
      for t in range(BSUB):
        if t >= 1 and t + NSLOT - 1 < BSUB:
          sds[t - 1].wait()
          gds.append(pltpu.async_copy(
              z_hbm.at[cbuf.at[pl.ds((t + NSLOT - 1) * 128, 128)]],
              slots.at[(t + NSLOT - 1) % NSLOT], gsem))
        gds[t].wait()
        sd = pltpu.make_async_copy(slots.at[t % NSLOT],
                                   acc.at[libuf.at[t]], ssem)
        sd.start(add=True)
        sds.append(sd)
      for t in range(max(BSUB - NSLOT, 0), BSUB):
        sds[t].wait()
      return carry

    lax.fori_loop(0, nblk, body, 0)

  @pl.when(c == 0)
  def _():
    edge_loop(zlo_hbm)

  @pl.when(c == 1)
  def _():
    edge_loop(zhi_hbm)

  plsc.subcore_barrier()

  def writeback(out_hbm, total):
    wds = {}
    segs = _segments(total, 128)
    for k, (off, seg) in enumerate(segs):
      if k >= 2:
        wds[k - 2].wait()
      pltpu.sync_copy(acc.at[pl.ds(s * WBF + off, seg)],
                      slots.at[k % 2].at[pl.ds(0, seg)])
      wds[k] = pltpu.async_copy(
          slots.at[k % 2].at[pl.ds(0, seg)],
          out_hbm.at[pl.ds(s * WBF + off, seg)], wsem)
    for k in range(max(len(segs) - 2, 0), len(segs)):
      wds[k].wait()

  @pl.when(c == 0)
  def _():
    @pl.when(s < 15)
    def _():
      writeback(outlo_hbm, WBF)

    @pl.when(s == 15)
    def _():
      writeback(outlo_hbm, WBF_LAST)

  @pl.when(c == 1)
  def _():
    @pl.when(s < 15)
    def _():
      writeback(outhi_hbm, WBF)

    @pl.when(s == 15)
    def _():
      writeback(outhi_hbm, WBF_LAST)


_spmm_call = pl.kernel(
    _spmm_body,
    out_type=[jax.ShapeDtypeStruct((N, DH), jnp.float32),
              jax.ShapeDtypeStruct((N, DH), jnp.float32)],
    mesh=_mesh,
    compiler_params=pltpu.CompilerParams(use_tc_tiling_on_sc=False),
    scratch_types=[
        pltpu.VMEM((EB,), jnp.int32),
        pltpu.VMEM((EB,), jnp.int32),
        pltpu.VMEM((BSUB, 128), jnp.int32),
        pltpu.VMEM((NSLOT, 128, DH), jnp.float32),
        pltpu.VMEM_SHARED((ACC_ROWS, DH), jnp.float32),
        pltpu.SemaphoreType.DMA,
        pltpu.SemaphoreType.DMA,
        pltpu.SemaphoreType.DMA,
        pltpu.SemaphoreType.DMA,
    ],
)


BR = 2000
GRID = N // BR


def _prep_body(x_ref, p0_ref, p1_ref, zlo_ref, zhi_ref, dis_ref):
  deg = p0_ref[...] + p1_ref[...]
  dis = jnp.where(deg > 0, lax.rsqrt(deg), jnp.float32(0.0))
  dis_ref[...] = dis
  z = dis * x_ref[...]
  zlo_ref[...] = z[:, :DH]
  zhi_ref[...] = z[:, DH:]


_prep_call = pl.pallas_call(
    _prep_body,
    grid=(GRID,),
    in_specs=[
        pl.BlockSpec((BR, D), lambda i: (i, 0)),
        pl.BlockSpec((BR, 1), lambda i: (i, 0)),
        pl.BlockSpec((BR, 1), lambda i: (i, 0)),
    ],
    out_specs=[
        pl.BlockSpec((BR, DH), lambda i: (i, 0)),
        pl.BlockSpec((BR, DH), lambda i: (i, 0)),
        pl.BlockSpec((BR, 1), lambda i: (i, 0)),
    ],
    out_shape=[
        jax.ShapeDtypeStruct((N, DH), jnp.float32),
        jax.ShapeDtypeStruct((N, DH), jnp.float32),
        jax.ShapeDtypeStruct((N, 1), jnp.float32),
    ],
)


def _scale2_body(slo_ref, shi_ref, dis_ref, zlo_ref, zhi_ref):
  dis = dis_ref[...]
  m = -(dis * dis)
  zlo_ref[...] = m * slo_ref[...]
  zhi_ref[...] = m * shi_ref[...]


_scale2_call = pl.pallas_call(
    _scale2_body,
    grid=(GRID,),
    in_specs=[
        pl.BlockSpec((BR, DH), lambda i: (i, 0)),
        pl.BlockSpec((BR, DH), lambda i: (i, 0)),
        pl.BlockSpec((BR, 1), lambda i: (i, 0)),
    ],
    out_specs=[
        pl.BlockSpec((BR, DH), lambda i: (i, 0)),
        pl.BlockSpec((BR, DH), lambda i: (i, 0)),
    ],
    out_shape=[
        jax.ShapeDtypeStruct((N, DH), jnp.float32),
        jax.ShapeDtypeStruct((N, DH), jnp.float32),
    ],
)


def _final_body(x_ref, s1lo_ref, s1hi_ref, s2lo_ref, s2hi_ref,
                dis_ref, w_ref, b_ref, o_ref):
  x = x_ref[...]
  dis = dis_ref[...]
  s1 = jnp.concatenate([s1lo_ref[...], s1hi_ref[...]], axis=1)
  s2 = jnp.concatenate([s2lo_ref[...], s2hi_ref[...]], axis=1)
  tx1 = -dis * s1
  tx2 = jnp.float32(-2.0) * dis * s2 - x
  w = w_ref[...]
  o = jnp.dot(x, w[0], preferred_element_type=jnp.float32)
  o += jnp.dot(tx1, w[1], preferred_element_type=jnp.float32)
  o += jnp.dot(tx2, w[2], preferred_element_type=jnp.float32)
  o_ref[...] = o + b_ref[...]


_final_call = pl.pallas_call(
    _final_body,
    grid=(GRID,),
    in_specs=[
        pl.BlockSpec((BR, D), lambda i: (i, 0)),
        pl.BlockSpec((BR, DH), lambda i: (i, 0)),
        pl.BlockSpec((BR, DH), lambda i: (i, 0)),
        pl.BlockSpec((BR, DH), lambda i: (i, 0)),
        pl.BlockSpec((BR, DH), lambda i: (i, 0)),
        pl.BlockSpec((BR, 1), lambda i: (i, 0)),
        pl.BlockSpec((3, D, G), lambda i: (0, 0, 0)),
        pl.BlockSpec((1, G), lambda i: (0, 0)),
    ],
    out_specs=pl.BlockSpec((BR, G), lambda i: (i, 0)),
    out_shape=jax.ShapeDtypeStruct((N, G), jnp.float32),
)


@jax.jit
def kernel(x, edge_index, weight, bias):
  n, h, f, q = x.shape
  g = weight.shape[-1]
  x2d = x.reshape(n, h * f * q)
  w3 = weight.reshape(weight.shape[0], h * f, g)
  b2 = bias.reshape(1, g)

  zrow = jnp.zeros((WBF,), jnp.float32)
  zblk = jnp.zeros((128, DH), jnp.float32)

  parts = _deg_call(edge_index, zrow)
  p0 = parts[:N].reshape(n, 1)
  p1 = parts[N:].reshape(n, 1)
  z1lo, z1hi, dis = _prep_call(x2d, p0, p1)
  s1lo, s1hi = _spmm_call(edge_index, z1lo, z1hi, zblk)
  z2lo, z2hi = _scale2_call(s1lo, s1hi, dis)
  s2lo, s2hi = _spmm_call(edge_index, z2lo, z2hi, zblk)
  out = _final_call(x2d, s1lo, s1hi, s2lo, s2hi, dis, w3, b2)
  return out.reshape(n, 1, g, q)


# speedup vs baseline: 252.0648x; 1.0532x over previous
import jax
import jax.numpy as jnp
from jax import lax
from jax.experimental import pallas as pl
from jax.experimental.pallas import tpu as pltpu
from jax.experimental.pallas import tpu_sc as plsc

N = 50000
E = 800000
D = 64
DH = D // 2
G = 16
NC = 2
NS = 16
L = 16

EB = 1280
BSUB = EB // 128
NBLK = E // EB
NSLOT = 6

DUMP = N
ACC_ROWS = N + 8
WBF = 3136
WBF_LAST = N - 15 * WBF

CE = 1280
NSUB = CE // 128
NCHUNK = E // CE

_mesh = plsc.VectorSubcoreMesh(
    core_axis_name="c", subcore_axis_name="s", num_cores=NC, num_subcores=NS)


def _segments(total, step):
  segs = []
  off = 0
  while off < total:
    seg = min(step, total - off)
    segs.append((off, seg))
    off += seg
  return segs


def _deg_body(edge_hbm, zrow_hbm, out_hbm, rbuf, cbuf, vbuf, zv, acc, sem):
  c = lax.axis_index("c")
  s = lax.axis_index("s")

  pltpu.sync_copy(zrow_hbm, zv)

  @pl.when(s < 15)
  def _():
    pltpu.sync_copy(zv, acc.at[pl.ds(s * WBF, WBF)])

  @pl.when(s == 15)
  def _():
    pltpu.sync_copy(zv.at[pl.ds(0, WBF_LAST)],
                    acc.at[pl.ds(s * WBF, WBF_LAST)])

  plsc.subcore_barrier()

  w = c * NS + s
  nck = (NCHUNK - w + NC * NS - 1) // (NC * NS)

  def body(j, carry):
    ck = w + j * (NC * NS)
    off = ck * CE
    descs = []
    for t in range(NSUB):
      descs.append(pltpu.async_copy(
          edge_hbm.at[0, pl.ds(off + t * 128, 128)], rbuf.at[t], sem))
    descs.append(pltpu.async_copy(edge_hbm.at[1, pl.ds(off, CE)], cbuf, sem))
    for d in descs:
      d.wait()
    for i in range(CE // L):
      r16 = rbuf[i // 8, pl.ds((i % 8) * L, L)]
      c16 = cbuf[pl.ds(i * L, L)]
      v16 = jnp.where(r16 != c16, jnp.float32(1.0), jnp.float32(0.0))
      vbuf[i // 8, pl.ds((i % 8) * L, L)] = v16
    for t in range(NSUB):
      pltpu.sync_copy(vbuf.at[t], acc.at[rbuf.at[t]], add=True)
    return carry

  lax.fori_loop(0, nck, body, 0)
  plsc.subcore_barrier()

  @pl.when(s < 15)
  def _():
    pltpu.sync_copy(acc.at[pl.ds(s * WBF, WBF)], zv)
    pltpu.sync_copy(zv, out_hbm.at[pl.ds(c * N + s * WBF, WBF)])

  @pl.when(s == 15)
  def _():
    pltpu.sync_copy(acc.at[pl.ds(s * WBF, WBF_LAST)],
                    zv.at[pl.ds(0, WBF_LAST)])
    pltpu.sync_copy(zv.at[pl.ds(0, WBF_LAST)],
                    out_hbm.at[pl.ds(c * N + s * WBF, WBF_LAST)])


_deg_call = pl.kernel(
    _deg_body,
    out_type=jax.ShapeDtypeStruct((NC * N,), jnp.float32),
    mesh=_mesh,
    scratch_types=[
        pltpu.VMEM((NSUB, 128), jnp.int32),
        pltpu.VMEM((CE,), jnp.int32),
        pltpu.VMEM((NSUB, 128), jnp.float32),
        pltpu.VMEM((WBF,), jnp.float32),
        pltpu.VMEM_SHARED((N,), jnp.float32),
        pltpu.SemaphoreType.DMA,
    ],
)


def _spmm_body(edge_hbm, zlo_hbm, zhi_hbm, zblk_hbm, outlo_hbm, outhi_hbm,
               rbuf, cbuf, libuf, slots, acc, esem, gsem, ssem, wsem):
  c = lax.axis_index("c")
  s = lax.axis_index("s")

  pltpu.sync_copy(zblk_hbm, slots.at[0])

  def zero_fill(total):
    zds = []
    for off, seg in _segments(total, 128):
      zds.append(pltpu.async_copy(slots.at[0].at[pl.ds(0, seg)],
                                  acc.at[pl.ds(s * WBF + off, seg)], wsem))
    for d in zds:
      d.wait()

  @pl.when(s < 15)
  def _():
    zero_fill(WBF)

  @pl.when(s == 15)
  def _():
    zero_fill(WBF_LAST)

  @pl.when(s == 0)
  def _():
    pltpu.sync_copy(slots.at[0].at[pl.ds(0, ACC_ROWS - N)],
                    acc.at[pl.ds(N, ACC_ROWS - N)])
  plsc.subcore_barrier()

  nblk = (NBLK - s + NS - 1) // NS

  def edge_loop(z_hbm):
    def body(j, carry):
      blk = s + j * NS
      off = blk * EB
      d0 = pltpu.async_copy(edge_hbm.at[0, pl.ds(off, EB)], rbuf, esem)
      d1 = pltpu.async_copy(edge_hbm.at[1, pl.ds(off, EB)], cbuf, esem)
      d0.wait()
      d1.wait()
      gds, sds = [], []
      for t in range(NSLOT):
        gds.append(pltpu.async_copy(
            z_hbm.at[cbuf.at[pl.ds(t * 128, 128)]], slots.at[t], gsem))
      for i in range(EB // L):
        r16 = rbuf[pl.ds(i * L, L)]
        c16 = cbuf[pl.ds(i * L, L)]
        li = jnp.where(r16 != c16, r16, jnp.int32(DUMP))
        libuf[i // 8, pl.ds((i % 8) * L, L)] = li
      for t in range(BSUB):
        if t >= 1 and t + NSLOT - 1 < BSUB:
          sds[t - 1].wait()
          gds.append(pltpu.async_copy(
              z_hbm.at[cbuf.at[pl.ds((t + NSLOT - 1) * 128, 128)]],
              slots.at[(t + NSLOT - 1) % NSLOT], gsem))
        gds[t].wait()
        sd = pltpu.make_async_copy(slots.at[t % NSLOT],
                                   acc.at[libuf.at[t]], ssem)
        sd.start(add=True)
        sds.append(sd)
      for t in range(max(BSUB - NSLOT, 0), BSUB):
        sds[t].wait()
      return carry

    lax.fori_loop(0, nblk, body, 0)

  @pl.when(c == 0)
  def _():
    edge_loop(zlo_hbm)

  @pl.when(c == 1)
  def _():
    edge_loop(zhi_hbm)

  plsc.subcore_barrier()

  def writeback(out_hbm, total):
    wds = {}
    segs = _segments(total, 128)
    for k, (off, seg) in enumerate(segs):
      if k >= 2:
        wds[k - 2].wait()
      pltpu.sync_copy(acc.at[pl.ds(s * WBF + off, seg)],
                      slots.at[k % 2].at[pl.ds(0, seg)])
      wds[k] = pltpu.async_copy(
          slots.at[k % 2].at[pl.ds(0, seg)],
          out_hbm.at[pl.ds(s * WBF + off, seg)], wsem)
    for k in range(max(len(segs) - 2, 0), len(segs)):
      wds[k].wait()

  @pl.when(c == 0)
  def _():
    @pl.when(s < 15)
    def _():
      writeback(outlo_hbm, WBF)

    @pl.when(s == 15)
    def _():
      writeback(outlo_hbm, WBF_LAST)

  @pl.when(c == 1)
  def _():
    @pl.when(s < 15)
    def _():
      writeback(outhi_hbm, WBF)

    @pl.when(s == 15)
    def _():
      writeback(outhi_hbm, WBF_LAST)


_spmm_call = pl.kernel(
    _spmm_body,
    out_type=[jax.ShapeDtypeStruct((N, DH), jnp.float32),
              jax.ShapeDtypeStruct((N, DH), jnp.float32)],
    mesh=_mesh,
    compiler_params=pltpu.CompilerParams(use_tc_tiling_on_sc=False),
    scratch_types=[
        pltpu.VMEM((EB,), jnp.int32),
        pltpu.VMEM((EB,), jnp.int32),
        pltpu.VMEM((BSUB, 128), jnp.int32),
        pltpu.VMEM((NSLOT, 128, DH), jnp.float32),
        pltpu.VMEM_SHARED((ACC_ROWS, DH), jnp.float32),
        pltpu.SemaphoreType.DMA,
        pltpu.SemaphoreType.DMA,
        pltpu.SemaphoreType.DMA,
        pltpu.SemaphoreType.DMA,
    ],
)


BR = 2000
GRID = N // BR


def _prep_body(x_ref, p0_ref, p1_ref, zlo_ref, zhi_ref, dis_ref):
  deg = p0_ref[...] + p1_ref[...]
  dis = jnp.where(deg > 0, lax.rsqrt(deg), jnp.float32(0.0))
  dis_ref[...] = dis
  z = dis * x_ref[...]
  zlo_ref[...] = z[:, :DH]
  zhi_ref[...] = z[:, DH:]


_prep_call = pl.pallas_call(
    _prep_body,
    grid=(GRID,),
    in_specs=[
        pl.BlockSpec((BR, D), lambda i: (i, 0)),
        pl.BlockSpec((BR, 1), lambda i: (i, 0)),
        pl.BlockSpec((BR, 1), lambda i: (i, 0)),
    ],
    out_specs=[
        pl.BlockSpec((BR, DH), lambda i: (i, 0)),
        pl.BlockSpec((BR, DH), lambda i: (i, 0)),
        pl.BlockSpec((BR, 1), lambda i: (i, 0)),
    ],
    out_shape=[
        jax.ShapeDtypeStruct((N, DH), jnp.float32),
        jax.ShapeDtypeStruct((N, DH), jnp.float32),
        jax.ShapeDtypeStruct((N, 1), jnp.float32),
    ],
)


def _scale2_body(slo_ref, shi_ref, dis_ref, zlo_ref, zhi_ref):
  dis = dis_ref[...]
  m = -(dis * dis)
  zlo_ref[...] = m * slo_ref[...]
  zhi_ref[...] = m * shi_ref[...]


_scale2_call = pl.pallas_call(
    _scale2_body,
    grid=(GRID,),
    in_specs=[
        pl.BlockSpec((BR, DH), lambda i: (i, 0)),
        pl.BlockSpec((BR, DH), lambda i: (i, 0)),
        pl.BlockSpec((BR, 1), lambda i: (i, 0)),
    ],
    out_specs=[
        pl.BlockSpec((BR, DH), lambda i: (i, 0)),
        pl.BlockSpec((BR, DH), lambda i: (i, 0)),
    ],
    out_shape=[
        jax.ShapeDtypeStruct((N, DH), jnp.float32),
        jax.ShapeDtypeStruct((N, DH), jnp.float32),
    ],
)


def _final_body(x_ref, s1lo_ref, s1hi_ref, s2lo_ref, s2hi_ref,
                dis_ref, w_ref, b_ref, o_ref):
  x = x_ref[...]
  dis = dis_ref[...]
  s1 = jnp.concatenate([s1lo_ref[...], s1hi_ref[...]], axis=1)
  s2 = jnp.concatenate([s2lo_ref[...], s2hi_ref[...]], axis=1)
  tx1 = -dis * s1
  tx2 = jnp.float32(-2.0) * dis * s2 - x
  w = w_ref[...]
  o = jnp.dot(x, w[0], preferred_element_type=jnp.float32)
  o += jnp.dot(tx1, w[1], preferred_element_type=jnp.float32)
  o += jnp.dot(tx2, w[2], preferred_element_type=jnp.float32)
  o_ref[...] = o + b_ref[...]


_final_call = pl.pallas_call(
    _final_body,
    grid=(GRID,),
    in_specs=[
        pl.BlockSpec((BR, D), lambda i: (i, 0)),
        pl.BlockSpec((BR, DH), lambda i: (i, 0)),
        pl.BlockSpec((BR, DH), lambda i: (i, 0)),
        pl.BlockSpec((BR, DH), lambda i: (i, 0)),
        pl.BlockSpec((BR, DH), lambda i: (i, 0)),
        pl.BlockSpec((BR, 1), lambda i: (i, 0)),
        pl.BlockSpec((3, D, G), lambda i: (0, 0, 0)),
        pl.BlockSpec((1, G), lambda i: (0, 0)),
    ],
    out_specs=pl.BlockSpec((BR, G), lambda i: (i, 0)),
    out_shape=jax.ShapeDtypeStruct((N, G), jnp.float32),
)


@jax.jit
def kernel(x, edge_index, weight, bias):
  n, h, f, q = x.shape
  g = weight.shape[-1]
  x2d = x.reshape(n, h * f * q)
  w3 = weight.reshape(weight.shape[0], h * f, g)
  b2 = bias.reshape(1, g)

  zrow = jnp.zeros((WBF,), jnp.float32)
  zblk = jnp.zeros((128, DH), jnp.float32)

  parts = _deg_call(edge_index, zrow)
  p0 = parts[:N].reshape(n, 1)
  p1 = parts[N:].reshape(n, 1)
  z1lo, z1hi, dis = _prep_call(x2d, p0, p1)
  s1lo, s1hi = _spmm_call(edge_index, z1lo, z1hi, zblk)
  z2lo, z2hi = _scale2_call(s1lo, s1hi, dis)
  s2lo, s2hi = _spmm_call(edge_index, z2lo, z2hi, zblk)
  out = _final_call(x2d, s1lo, s1hi, s2lo, s2hi, dis, w3, b2)
  return out.reshape(n, 1, g, q)


# speedup vs baseline: 268.1558x; 1.0638x over previous
import jax
import jax.numpy as jnp
from jax import lax
from jax.experimental import pallas as pl
from jax.experimental.pallas import tpu as pltpu
from jax.experimental.pallas import tpu_sc as plsc

N = 50000
E = 800000
D = 64
DH = D // 2
G = 16
NC = 2
NS = 16
L = 16

EB = 1280
BSUB = EB // 128
NBLK = E // EB
NSLOT = 6

DUMP = N
ACC_ROWS = N + 8
WBF = 3136
WBF_LAST = N - 15 * WBF

CE = 1280
NSUB = CE // 128
NCHUNK = E // CE

_mesh = plsc.VectorSubcoreMesh(
    core_axis_name="c", subcore_axis_name="s", num_cores=NC, num_subcores=NS)


def _segments(total, step):
  segs = []
  off = 0
  while off < total:
    seg = min(step, total - off)
    segs.append((off, seg))
    off += seg
  return segs


def _deg_body(edge_hbm, zrow_hbm, out_hbm, rbuf, cbuf, vbuf, zv, acc, sem):
  c = lax.axis_index("c")
  s = lax.axis_index("s")

  pltpu.sync_copy(zrow_hbm, zv)

  @pl.when(s < 15)
  def _():
    pltpu.sync_copy(zv, acc.at[pl.ds(s * WBF, WBF)])

  @pl.when(s == 15)
  def _():
    pltpu.sync_copy(zv.at[pl.ds(0, WBF_LAST)],
                    acc.at[pl.ds(s * WBF, WBF_LAST)])

  plsc.subcore_barrier()

  w = c * NS + s
  nck = (NCHUNK - w + NC * NS - 1) // (NC * NS)

  def body(j, carry):
    ck = w + j * (NC * NS)
    off = ck * CE
    descs = []
    for t in range(NSUB):
      descs.append(pltpu.async_copy(
          edge_hbm.at[0, pl.ds(off + t * 128, 128)], rbuf.at[t], sem))
    descs.append(pltpu.async_copy(edge_hbm.at[1, pl.ds(off, CE)], cbuf, sem))
    for d in descs:
      d.wait()
    for i in range(CE // L):
      r16 = rbuf[i // 8, pl.ds((i % 8) * L, L)]
      c16 = cbuf[pl.ds(i * L, L)]
      v16 = jnp.where(r16 != c16, jnp.float32(1.0), jnp.float32(0.0))
      vbuf[i // 8, pl.ds((i % 8) * L, L)] = v16
    for t in range(NSUB):
      pltpu.sync_copy(vbuf.at[t], acc.at[rbuf.at[t]], add=True)
    return carry

  lax.fori_loop(0, nck, body, 0)
  plsc.subcore_barrier()

  @pl.when(s < 15)
  def _():
    pltpu.sync_copy(acc.at[pl.ds(s * WBF, WBF)], zv)
    pltpu.sync_copy(zv, out_hbm.at[pl.ds(c * N + s * WBF, WBF)])

  @pl.when(s == 15)
  def _():
    pltpu.sync_copy(acc.at[pl.ds(s * WBF, WBF_LAST)],
                    zv.at[pl.ds(0, WBF_LAST)])
    pltpu.sync_copy(zv.at[pl.ds(0, WBF_LAST)],
                    out_hbm.at[pl.ds(c * N + s * WBF, WBF_LAST)])


_deg_call = pl.kernel(
    _deg_body,
    out_type=jax.ShapeDtypeStruct((NC * N,), jnp.float32),
    mesh=_mesh,
    scratch_types=[
        pltpu.VMEM((NSUB, 128), jnp.int32),
        pltpu.VMEM((CE,), jnp.int32),
        pltpu.VMEM((NSUB, 128), jnp.float32),
        pltpu.VMEM((WBF,), jnp.float32),
        pltpu.VMEM_SHARED((N,), jnp.float32),
        pltpu.SemaphoreType.DMA,
    ],
)


def _spmm_body(edge_hbm, zlo_hbm, zhi_hbm, dis_hbm, zblk_hbm,
               s1lo_hbm, s1hi_hbm, s2lo_hbm, s2hi_hbm, z2lo_hbm, z2hi_hbm,
               rbuf, cbuf, libuf, dbuf, slots, acc, esem, gsem, ssem, wsem):
  c = lax.axis_index("c")
  s = lax.axis_index("s")

  pltpu.sync_copy(zblk_hbm, slots.at[0])

  def zero_fill(total):
    zds = []
    for off, seg in _segments(total, 128):
      zds.append(pltpu.async_copy(slots.at[0].at[pl.ds(0, seg)],
                                  acc.at[pl.ds(s * WBF + off, seg)], wsem))
    for d in zds:
      d.wait()

  @pl.when(s < 15)
  def _():
    zero_fill(WBF)

  @pl.when(s == 15)
  def _():
    zero_fill(WBF_LAST)

  @pl.when(s == 0)
  def _():
    pltpu.sync_copy(slots.at[0].at[pl.ds(0, ACC_ROWS - N)],
                    acc.at[pl.ds(N, ACC_ROWS - N)])
  plsc.subcore_barrier()

  nblk = (NBLK - s + NS - 1) // NS

  def edge_loop(z_hbm):
    def body(j, carry):
      blk = s + j * NS
      off = blk * EB
      d0 = pltpu.async_copy(edge_hbm.at[0, pl.ds(off, EB)], rbuf, esem)
      d1 = pltpu.async_copy(edge_hbm.at[1, pl.ds(off, EB)], cbuf, esem)
      d0.wait()
      d1.wait()
      gds, sds = [], []
      for t in range(NSLOT):
        gds.append(pltpu.async_copy(
            z_hbm.at[cbuf.at[pl.ds(t * 128, 128)]], slots.at[t], gsem))
      for i in range(EB // L):
        r16 = rbuf[pl.ds(i * L, L)]
        c16 = cbuf[pl.ds(i * L, L)]
        li = jnp.where(r16 != c16, r16, jnp.int32(DUMP))
        libuf[i // 8, pl.ds((i % 8) * L, L)] = li
      for t in range(BSUB):
        if t >= 1 and t + NSLOT - 1 < BSUB:
          sds[t - 1].wait()
          gds.append(pltpu.async_copy(
              z_hbm.at[cbuf.at[pl.ds((t + NSLOT - 1) * 128, 128)]],
              slots.at[(t + NSLOT - 1) % NSLOT], gsem))
        gds[t].wait()
        sd = pltpu.make_async_copy(slots.at[t % NSLOT],
                                   acc.at[libuf.at[t]], ssem)
        sd.start(add=True)
        sds.append(sd)
      for t in range(max(BSUB - NSLOT, 0), BSUB):
        sds[t].wait()
      return carry

    lax.fori_loop(0, nblk, body, 0)

  @pl.when(c == 0)
  def _():
    edge_loop(zlo_hbm)

  @pl.when(c == 1)
  def _():
    edge_loop(zhi_hbm)

  plsc.subcore_barrier()

  st0 = slots.at[0]
  st1 = slots.at[1]

  def wb_s1(s1out_hbm, z2out_hbm, total):
    nfull = total // 128
    tail = total - nfull * 128

    def seg(k, carry):
      off = s * WBF + k * 128

      @pl.when(k >= 1)
      def _():
        prev = s * WBF + (k - 1) * 128
        pltpu.make_async_copy(st0, s1out_hbm.at[pl.ds(prev, 128)],
                              wsem).wait()
        pltpu.make_async_copy(st1, z2out_hbm.at[pl.ds(prev, 128)],
                              wsem).wait()
      pltpu.sync_copy(acc.at[pl.ds(off, 128)], st0)
      pltpu.sync_copy(dis_hbm.at[pl.ds(off, 128)], dbuf)
      for g in range(8):
        dv = dbuf[pl.ds(g * L, L)]
        mv = -(dv * dv)
        for r16 in range(L):
          r = g * L + r16
          for hh in range(2):
            st1[r, pl.ds(hh * L, L)] = st0[r, pl.ds(hh * L, L)] * mv[r16]
      pltpu.async_copy(st0, s1out_hbm.at[pl.ds(off, 128)], wsem)
      pltpu.async_copy(st1, z2out_hbm.at[pl.ds(off, 128)], wsem)
      return carry

    lax.fori_loop(0, nfull, seg, 0)
    pltpu.make_async_copy(zblk_hbm, slots.at[5], wsem).wait()
    pltpu.make_async_copy(zblk_hbm, slots.at[5], wsem).wait()
    toff = s * WBF + nfull * 128
    pltpu.sync_copy(acc.at[pl.ds(toff, tail)], st0.at[pl.ds(0, tail)])
    pltpu.sync_copy(dis_hbm.at[pl.ds(toff, tail)], dbuf.at[pl.ds(0, tail)])
    for g in range(tail // L):
      dv = dbuf[pl.ds(g * L, L)]
      mv = -(dv * dv)
      for r16 in range(L):
        r = g * L + r16
        for hh in range(2):
          st1[r, pl.ds(hh * L, L)] = st0[r, pl.ds(hh * L, L)] * mv[r16]
    pltpu.sync_copy(st0.at[pl.ds(0, tail)],
                    s1out_hbm.at[pl.ds(toff, tail)])
    pltpu.sync_copy(st1.at[pl.ds(0, tail)],
                    z2out_hbm.at[pl.ds(toff, tail)])

  @pl.when(c == 0)
  def _():
    @pl.when(s < 15)
    def _():
      wb_s1(s1lo_hbm, z2lo_hbm, WBF)

    @pl.when(s == 15)
    def _():
      wb_s1(s1lo_hbm, z2lo_hbm, WBF_LAST)

  @pl.when(c == 1)
  def _():
    @pl.when(s < 15)
    def _():
      wb_s1(s1hi_hbm, z2hi_hbm, WBF)

    @pl.when(s == 15)
    def _():
      wb_s1(s1hi_hbm, z2hi_hbm, WBF_LAST)

  pltpu.sync_copy(zblk_hbm, slots.at[0])

  @pl.when(s < 15)
  def _():
    zero_fill(WBF)

  @pl.when(s == 15)
  def _():
    zero_fill(WBF_LAST)

  @pl.when(s == 0)
  def _():
    pltpu.sync_copy(slots.at[0].at[pl.ds(0, ACC_ROWS - N)],
                    acc.at[pl.ds(N, ACC_ROWS - N)])
  plsc.subcore_barrier()

  @pl.when(c == 0)
  def _():
    edge_loop(z2lo_hbm)

  @pl.when(c == 1)
  def _():
    edge_loop(z2hi_hbm)

  plsc.subcore_barrier()

  def writeback(out_hbm, total):
    wds = {}
    segs = _segments(total, 128)
    for k, (off, seg) in enumerate(segs):
      if k >= 2:
        wds[k - 2].wait()
      pltpu.sync_copy(acc.at[pl.ds(s * WBF + off, seg)],
                      slots.at[k % 2].at[pl.ds(0, seg)])
      wds[k] = pltpu.async_copy(
          slots.at[k % 2].at[pl.ds(0, seg)],
          out_hbm.at[pl.ds(s * WBF + off, seg)], wsem)
    for k in range(max(len(segs) - 2, 0), len(segs)):
      wds[k].wait()

  @pl.when(c == 0)
  def _():
    @pl.when(s < 15)
    def _():
      writeback(s2lo_hbm, WBF)

    @pl.when(s == 15)
    def _():
      writeback(s2lo_hbm, WBF_LAST)

  @pl.when(c == 1)
  def _():
    @pl.when(s < 15)
    def _():
      writeback(s2hi_hbm, WBF)

    @pl.when(s == 15)
    def _():
      writeback(s2hi_hbm, WBF_LAST)


_spmm_call = pl.kernel(
    _spmm_body,
    out_type=[jax.ShapeDtypeStruct((N, DH), jnp.float32),
              jax.ShapeDtypeStruct((N, DH), jnp.float32),
              jax.ShapeDtypeStruct((N, DH), jnp.float32),
              jax.ShapeDtypeStruct((N, DH), jnp.float32),
              jax.ShapeDtypeStruct((N, DH), jnp.float32),
              jax.ShapeDtypeStruct((N, DH), jnp.float32)],
    mesh=_mesh,
    compiler_params=pltpu.CompilerParams(use_tc_tiling_on_sc=False),
    scratch_types=[
        pltpu.VMEM((EB,), jnp.int32),
        pltpu.VMEM((EB,), jnp.int32),
        pltpu.VMEM((BSUB, 128), jnp.int32),
        pltpu.VMEM((128,), jnp.float32),
        pltpu.VMEM((NSLOT, 128, DH), jnp.float32),
        pltpu.VMEM_SHARED((ACC_ROWS, DH), jnp.float32),
        pltpu.SemaphoreType.DMA,
        pltpu.SemaphoreType.DMA,
        pltpu.SemaphoreType.DMA,
        pltpu.SemaphoreType.DMA,
    ],
)


BR = 2000
GRID = N // BR


def _prep_body(x_ref, p0_ref, p1_ref, zlo_ref, zhi_ref, dis_ref):
  deg = p0_ref[...] + p1_ref[...]
  dis = jnp.where(deg > 0, lax.rsqrt(deg), jnp.float32(0.0))
  dis_ref[...] = dis
  z = dis * x_ref[...]
  zlo_ref[...] = z[:, :DH]
  zhi_ref[...] = z[:, DH:]


_prep_call = pl.pallas_call(
    _prep_body,
    grid=(GRID,),
    in_specs=[
        pl.BlockSpec((BR, D), lambda i: (i, 0)),
        pl.BlockSpec((BR, 1), lambda i: (i, 0)),
        pl.BlockSpec((BR, 1), lambda i: (i, 0)),
    ],
    out_specs=[
        pl.BlockSpec((BR, DH), lambda i: (i, 0)),
        pl.BlockSpec((BR, DH), lambda i: (i, 0)),
        pl.BlockSpec((BR, 1), lambda i: (i, 0)),
    ],
    out_shape=[
        jax.ShapeDtypeStruct((N, DH), jnp.float32),
        jax.ShapeDtypeStruct((N, DH), jnp.float32),
        jax.ShapeDtypeStruct((N, 1), jnp.float32),
    ],
)


def _final_body(x_ref, s1lo_ref, s1hi_ref, s2lo_ref, s2hi_ref,
                dis_ref, w_ref, b_ref, o_ref):
  x = x_ref[...]
  dis = dis_ref[...]
  s1 = jnp.concatenate([s1lo_ref[...], s1hi_ref[...]], axis=1)
  s2 = jnp.concatenate([s2lo_ref[...], s2hi_ref[...]], axis=1)
  tx1 = -dis * s1
  tx2 = jnp.float32(-2.0) * dis * s2 - x
  w = w_ref[...]
  o = jnp.dot(x, w[0], preferred_element_type=jnp.float32)
  o += jnp.dot(tx1, w[1], preferred_element_type=jnp.float32)
  o += jnp.dot(tx2, w[2], preferred_element_type=jnp.float32)
  o_ref[...] = o + b_ref[...]


_final_call = pl.pallas_call(
    _final_body,
    grid=(GRID,),
    in_specs=[
        pl.BlockSpec((BR, D), lambda i: (i, 0)),
        pl.BlockSpec((BR, DH), lambda i: (i, 0)),
        pl.BlockSpec((BR, DH), lambda i: (i, 0)),
        pl.BlockSpec((BR, DH), lambda i: (i, 0)),
        pl.BlockSpec((BR, DH), lambda i: (i, 0)),
        pl.BlockSpec((BR, 1), lambda i: (i, 0)),
        pl.BlockSpec((3, D, G), lambda i: (0, 0, 0)),
        pl.BlockSpec((1, G), lambda i: (0, 0)),
    ],
    out_specs=pl.BlockSpec((BR, G), lambda i: (i, 0)),
    out_shape=jax.ShapeDtypeStruct((N, G), jnp.float32),
)


@jax.jit
def kernel(x, edge_index, weight, bias):
  n, h, f, q = x.shape
  g = weight.shape[-1]
  x2d = x.reshape(n, h * f * q)
  w3 = weight.reshape(weight.shape[0], h * f, g)
  b2 = bias.reshape(1, g)

  zrow = jnp.zeros((WBF,), jnp.float32)
  zblk = jnp.zeros((128, DH), jnp.float32)

  parts = _deg_call(edge_index, zrow)
  p0 = parts[:N].reshape(n, 1)
  p1 = parts[N:].reshape(n, 1)
  z1lo, z1hi, dis = _prep_call(x2d, p0, p1)
  s1lo, s1hi, s2lo, s2hi, _, _ = _spmm_call(
      edge_index, z1lo, z1hi, dis.reshape(n), zblk)
  out = _final_call(x2d, s1lo, s1hi, s2lo, s2hi, dis, w3, b2)
  return out.reshape(n, 1, g, q)
